# baseline (speedup 1.0000x reference)
import functools

import jax
import jax.numpy as jnp
from jax import lax
from jax.experimental import pallas as pl
from jax.experimental.pallas import tpu as pltpu
from jax.experimental.pallas import tpu_sc as plsc

N_VERTS = 35709
ROW = N_VERTS * 3
BATCH = 128
NUM_WORKERS = 32
ROWS_PER_W = BATCH // NUM_WORKERS

REST, EYE, MOUTH, SPLIT, TAIL = 0, 1, 2, 3, 4
N_MOUTH = 1700 * 3 * BATCH
N_EYE = 1600 * 3 * BATCH
N_REST = (N_VERTS - 3300) * 3 * BATCH

CH = 24576

_SEGS = ((0, 12000, REST), (12000, 14400, EYE), (14400, 24000, REST),
         (24000, 26400, EYE), (26400, 38400, REST), (38400, 43488, MOUTH),
         (43488, 43504, SPLIT), (43504, 107120, REST), (107120, ROW, TAIL))


def _chunk_table():
    chunks = []
    cs = 0
    while cs < ROW:
        ce = min(cs + CH, ROW)
        spans = []
        for s, e, kind in _SEGS:
            lo, hi = max(s, cs), min(e, ce)
            if lo < hi:
                spans.append((lo - cs, hi - lo, kind))
        chunks.append((cs, ce - cs, tuple(spans)))
        cs = ce
    return tuple(chunks)

DMA_CHUNKS = _chunk_table()
UNROLL = 8


def _span_sum(xb, yb, off, nvec):
    a0 = jnp.zeros((16,), jnp.float32)
    a1 = jnp.zeros((16,), jnp.float32)
    n_u = nvec // UNROLL

    if n_u > 0:
        def body(i, accs):
            b0, b1 = accs
            base = off + i * (16 * UNROLL)
            for u in range(UNROLL):
                o = base + u * 16
                v = jnp.abs(xb[pl.ds(o, 16)] - yb[pl.ds(o, 16)])
                if u % 2 == 0:
                    b0 = b0 + v
                else:
                    b1 = b1 + v
            return (b0, b1)
        a0, a1 = lax.fori_loop(0, n_u, body, (a0, a1))
    base = off + n_u * (16 * UNROLL)
    for u in range(nvec % UNROLL):
        o = base + u * 16
        v = jnp.abs(xb[pl.ds(o, 16)] - yb[pl.ds(o, 16)])
        if u % 2 == 0:
            a0 = a0 + v
        else:
            a1 = a1 + v
    return a0 + a1


def _region_l1_sc(x_hbm, y_hbm, out_hbm, xbuf0, xbuf1, ybuf0, ybuf1,
                  accbuf, sem0, sem1):
    wid = lax.axis_index("s") * 2 + lax.axis_index("c")
    zero = jnp.zeros((16,), jnp.float32)
    lane = lax.iota(jnp.int32, 16)
    sems = (sem0, sem1)
    xbufs = (xbuf0, xbuf1)
    ybufs = (ybuf0, ybuf1)
    nchunks = len(DMA_CHUNKS)
    last = nchunks - 1

    def issue(frow, m, c, slot):
        cs, cl, _ = DMA_CHUNKS[c]
        extra = 16 if c == last else 0
        start = pl.multiple_of(frow + cs - m - extra, 16)
        hx = pltpu.async_copy(x_hbm.at[pl.ds(start, cl + 16)],
                              xbufs[slot].at[pl.ds(0, cl + 16)], sems[slot])
        hy = pltpu.async_copy(y_hbm.at[pl.ds(start, cl + 16)],
                              ybufs[slot].at[pl.ds(0, cl + 16)], sems[slot])
        return hx, hy

    def row_body(r, accs):
        row = wid * ROWS_PER_W + r
        frow = row * ROW
        m = (row * 7) & 15
        acc = list(accs)
        handles = [None, None]
        handles[0] = issue(frow, m, 0, 0)
        handles[1] = issue(frow, m, 1, 1)
        for c, (cs, cl, spans) in enumerate(DMA_CHUNKS):
            slot = c % 2
            hx, hy = handles[slot]
            hx.wait()
            hy.wait()
            xb, yb = xbufs[slot], ybufs[slot]
            mb = m + (16 if c == last else 0)
            for off, ln, kind in spans:
                o0 = mb + off
                if kind == SPLIT:
                    d = jnp.abs(xb[pl.ds(o0, 16)] - yb[pl.ds(o0, 16)])
                    acc[MOUTH] = acc[MOUTH] + jnp.where(lane < 12, d, 0.0)
                    acc[REST] = acc[REST] + jnp.where(lane >= 12, d, 0.0)
                elif kind == TAIL:
                    d = jnp.abs(xb[pl.ds(o0, 16)] - yb[pl.ds(o0, 16)])
                    acc[REST] = acc[REST] + jnp.where(lane < ln, d, 0.0)
                else:
                    acc[kind] = acc[kind] + _span_sum(xb, yb, o0, ln // 16)
            if c + 2 < nchunks:
                handles[slot] = issue(frow, m, c + 2, slot)
        return tuple(acc)

    acc_rest, acc_eye, acc_mouth = lax.fori_loop(
        0, ROWS_PER_W, row_body, (zero, zero, zero))
    accbuf[0, :] = acc_rest
    accbuf[1, :] = acc_eye
    accbuf[2, :] = acc_mouth
    pltpu.sync_copy(accbuf, out_hbm.at[wid])


@functools.cache
def _build_sc_kernel():
    mesh = plsc.VectorSubcoreMesh(core_axis_name="c", subcore_axis_name="s")
    return functools.partial(
        pl.kernel,
        mesh=mesh,
        out_type=jax.ShapeDtypeStruct((NUM_WORKERS, 3, 16), jnp.float32),
        scratch_types=[
            pltpu.VMEM((CH + 32,), jnp.float32),
            pltpu.VMEM((CH + 32,), jnp.float32),
            pltpu.VMEM((CH + 32,), jnp.float32),
            pltpu.VMEM((CH + 32,), jnp.float32),
            pltpu.VMEM((3, 16), jnp.float32),
            pltpu.SemaphoreType.DMA,
            pltpu.SemaphoreType.DMA,
        ],
        compiler_params=pltpu.CompilerParams(use_tc_tiling_on_sc=False),
    )(_region_l1_sc)


def kernel(input, target):
    x = input.reshape(BATCH * ROW)
    y = target.reshape(BATCH * ROW)
    partials = _build_sc_kernel()(x, y)
    sums = partials.sum(axis=(0, 2))
    mouth_loss = sums[MOUTH] / N_MOUTH
    eye_loss = sums[EYE] / N_EYE
    rest_loss = sums[REST] / N_REST
    return (mouth_loss, eye_loss, rest_loss)


# speedup vs baseline: 1.0029x; 1.0029x over previous
import functools

import jax
import jax.numpy as jnp
from jax import lax
from jax.experimental import pallas as pl
from jax.experimental.pallas import tpu as pltpu
from jax.experimental.pallas import tpu_sc as plsc

N_VERTS = 35709
ROW = N_VERTS * 3
BATCH = 128
TOTAL = BATCH * ROW
NUM_WORKERS = 32
ROWS_PER_W = BATCH // NUM_WORKERS

REST, EYE, MOUTH = 0, 1, 2
N_MOUTH = 1700 * 3 * BATCH
N_EYE = 1600 * 3 * BATCH
N_REST = (N_VERTS - 3300) * 3 * BATCH

CH = 24576
LAST_CL = 8816
UNROLL = 8

_SEGS = ((0, 12000, REST), (12000, 14400, EYE), (14400, 24000, REST),
         (24000, 26400, EYE), (26400, 38400, REST), (38400, 43500, MOUTH),
         (43500, ROW, REST))

_CHUNKS = ((0, CH), (CH, CH), (2 * CH, CH), (3 * CH, CH), (4 * CH, LAST_CL))

def _candidates(k):
    cs, cl = _CHUNKS[k]
    out = []
    for a, b, r in _SEGS:
        if a < cs + cl and b > cs - 15:
            out.append((a, b, r))
    return tuple(out)

_CAND = tuple(_candidates(k) for k in range(len(_CHUNKS)))


def _floor16(v):
    return v & -16


def _ceil16(v):
    return (v + 15) & -16


def _masked_add(xb, yb, base, lane, lo, hi, acc):
    base = pl.multiple_of(base, 16)
    d = jnp.abs(xb[pl.ds(base, 16)] - yb[pl.ds(base, 16)])
    msk = (lane >= lo) & (lane < hi)
    return acc + jnp.where(msk, d, 0.0)


def _bulk_sum(xb, yb, vlo, nvec):
    zero = jnp.zeros((16,), jnp.float32)
    vlo = pl.multiple_of(vlo, 16)
    n8 = nvec // UNROLL

    def body8(i, accs):
        b0, b1 = accs
        base = pl.multiple_of(vlo + i * (16 * UNROLL), 16)
        for u in range(UNROLL):
            o = pl.multiple_of(base + u * 16, 16)
            v = jnp.abs(xb[pl.ds(o, 16)] - yb[pl.ds(o, 16)])
            if u % 2 == 0:
                b0 = b0 + v
            else:
                b1 = b1 + v
        return (b0, b1)

    a0, a1 = lax.fori_loop(0, n8, body8, (zero, zero))
    rem_base = vlo + n8 * (16 * UNROLL)

    def body1(i, a):
        o = pl.multiple_of(rem_base + i * 16, 16)
        return a + jnp.abs(xb[pl.ds(o, 16)] - yb[pl.ds(o, 16)])

    a0 = lax.fori_loop(0, nvec - n8 * UNROLL, body1, a0)
    return a0 + a1


def _span(xb, yb, lane, wstart, cl, a, b, acc):
    ja = jnp.clip(a - wstart, 0, cl)
    jb = jnp.clip(b - wstart, 0, cl)
    vlo = _ceil16(ja)
    vhi = _floor16(jb)
    hb = _floor16(ja)
    acc = _masked_add(xb, yb, hb, lane, ja - hb, jnp.minimum(jb, vlo) - hb,
                      acc)
    nvec = jnp.maximum(vhi - vlo, 0) // 16
    acc = acc + _bulk_sum(xb, yb, vlo, nvec)
    tlo = jnp.maximum(jnp.maximum(ja, vlo), vhi)
    acc = _masked_add(xb, yb, vhi, lane, tlo - vhi, jb - vhi, acc)
    return acc


def _region_l1_sc(x_hbm, y_hbm, out_hbm, xbuf0, xbuf1, ybuf0, ybuf1,
                  xt0, xt1, yt0, yt1, accbuf, sem0, sem1, semt):
    wid = lax.axis_index("s") * 2 + lax.axis_index("c")
    zero = jnp.zeros((16,), jnp.float32)
    lane = lax.iota(jnp.int32, 16)
    sems = (sem0, sem1)
    xbufs = (xbuf0, xbuf1)
    ybufs = (ybuf0, ybuf1)
    nchunks = len(_CHUNKS)

    def issue(albase, c, slot):
        cs, cl = _CHUNKS[c]
        start = pl.multiple_of(albase + cs, 16)
        hx = pltpu.async_copy(x_hbm.at[pl.ds(start, cl)],
                              xbufs[slot].at[pl.ds(0, cl)], sems[slot])
        hy = pltpu.async_copy(y_hbm.at[pl.ds(start, cl)],
                              ybufs[slot].at[pl.ds(0, cl)], sems[slot])
        return hx, hy

    def row_body(r, accs):
        row = wid * ROWS_PER_W + r
        frow = row * ROW
        m = frow & 15
        albase = pl.multiple_of(frow - m, 16)
        acc = list(accs)

        t0a = pl.multiple_of(albase + 4 * CH + LAST_CL, 16)
        t1a = pl.multiple_of(
            jnp.minimum(albase + 4 * CH + LAST_CL + 16, TOTAL - 16), 16)
        ht = [pltpu.async_copy(x_hbm.at[pl.ds(t0a, 16)], xt0, semt),
              pltpu.async_copy(y_hbm.at[pl.ds(t0a, 16)], yt0, semt),
              pltpu.async_copy(x_hbm.at[pl.ds(t1a, 16)], xt1, semt),
              pltpu.async_copy(y_hbm.at[pl.ds(t1a, 16)], yt1, semt)]

        handles = [None, None]
        handles[0] = issue(albase, 0, 0)
        handles[1] = issue(albase, 1, 1)
        for c in range(nchunks):
            cs, cl = _CHUNKS[c]
            slot = c % 2
            hx, hy = handles[slot]
            hx.wait()
            hy.wait()
            xb, yb = xbufs[slot], ybufs[slot]
            wstart = cs - m
            for a, b, reg in _CAND[c]:
                acc[reg] = _span(xb, yb, lane, wstart, cl, a, b, acc[reg])
            if c + 2 < nchunks:
                handles[slot] = issue(albase, c + 2, slot)

        for h in ht:
            h.wait()
        acc[REST] = _masked_add(xt0, yt0, 0, lane, 0,
                                jnp.minimum(m + 7, 16), acc[REST])
        acc[REST] = _masked_add(xt1, yt1, 0, lane, 0, m - 9, acc[REST])
        return tuple(acc)

    acc_rest, acc_eye, acc_mouth = lax.fori_loop(
        0, ROWS_PER_W, row_body, (zero, zero, zero))
    accbuf[0, :] = acc_rest
    accbuf[1, :] = acc_eye
    accbuf[2, :] = acc_mouth
    pltpu.sync_copy(accbuf, out_hbm.at[wid])


@functools.cache
def _build_sc_kernel():
    mesh = plsc.VectorSubcoreMesh(core_axis_name="c", subcore_axis_name="s")
    return functools.partial(
        pl.kernel,
        mesh=mesh,
        out_type=jax.ShapeDtypeStruct((NUM_WORKERS, 3, 16), jnp.float32),
        scratch_types=[
            pltpu.VMEM((CH + 16,), jnp.float32),
            pltpu.VMEM((CH + 16,), jnp.float32),
            pltpu.VMEM((CH + 16,), jnp.float32),
            pltpu.VMEM((CH + 16,), jnp.float32),
            pltpu.VMEM((16,), jnp.float32),
            pltpu.VMEM((16,), jnp.float32),
            pltpu.VMEM((16,), jnp.float32),
            pltpu.VMEM((16,), jnp.float32),
            pltpu.VMEM((3, 16), jnp.float32),
            pltpu.SemaphoreType.DMA,
            pltpu.SemaphoreType.DMA,
            pltpu.SemaphoreType.DMA,
        ],
        compiler_params=pltpu.CompilerParams(use_tc_tiling_on_sc=False),
    )(_region_l1_sc)


def kernel(input, target):
    x = input.reshape(TOTAL)
    y = target.reshape(TOTAL)
    partials = _build_sc_kernel()(x, y)
    sums = partials.sum(axis=(0, 2))
    mouth_loss = sums[MOUTH] / N_MOUTH
    eye_loss = sums[EYE] / N_EYE
    rest_loss = sums[REST] / N_REST
    return (mouth_loss, eye_loss, rest_loss)
